# baseline (speedup 1.0000x reference)
import functools

import jax
import jax.numpy as jnp
from jax import lax
from jax.experimental import pallas as pl
from jax.experimental.pallas import tpu as pltpu
from jax.experimental.pallas import tpu_sc as plsc

CHUNK = 32


def kernel(x, pe):
    batch, seq_len, dim = x.shape
    info = plsc.get_sparse_core_info()
    n_workers = info.num_cores * info.num_subcores
    s_per_w = seq_len // n_workers
    mesh = plsc.VectorSubcoreMesh(core_axis_name="c", subcore_axis_name="s")

    @functools.partial(
        pl.kernel,
        mesh=mesh,
        out_type=jax.ShapeDtypeStruct((batch, seq_len, dim), x.dtype),
        scratch_types=[
            pltpu.VMEM((CHUNK, dim), jnp.float32),
            pltpu.VMEM((CHUNK, dim), jnp.float32),
        ],
    )
    def sc_add(x_hbm, pe_hbm, out_hbm, xbuf, pbuf):
        wid = lax.axis_index("s") * info.num_cores + lax.axis_index("c")
        base = wid * s_per_w

        def row_add(r, carry):
            for j in range(dim // 16):
                sl = pl.ds(j * 16, 16)
                xbuf[r, sl] = xbuf[r, sl] + pbuf[r, sl]
            return carry

        for c in range(s_per_w // CHUNK):
            start = base + c * CHUNK
            pltpu.sync_copy(pe_hbm.at[pl.ds(start, CHUNK), :], pbuf)
            for b in range(batch):
                pltpu.sync_copy(x_hbm.at[b, pl.ds(start, CHUNK), :], xbuf)
                lax.fori_loop(0, CHUNK, row_add, 0)
                pltpu.sync_copy(xbuf, out_hbm.at[b, pl.ds(start, CHUNK), :])

    return sc_add(x, pe[:seq_len])
